# Initial kernel scaffold; baseline (speedup 1.0000x reference)
import jax
import jax.numpy as jnp
from jax.experimental import pallas as pl


def kernel(x_atoms, x_bonds, ei_a2b_src, ei_a2b_dst, ei_b2a_src, ei_b2a_dst, W_ua, b_ua, W_ub, b_ub, Wn_b0, Wr_b0, bb0, Wn_a0, Wr_a0, ba0, Wn_b1, Wr_b1, bb1, Wn_a1, Wr_a1, ba1, Wn_b2, Wr_b2, bb2, Wn_a2, Wr_a2, ba2, Wp1, bp1, Wp2, bp2):
    raise NotImplementedError("write your pallas kernel here")


# speedup vs baseline: 1.1491x; 1.1491x over previous
import functools

import jax
import jax.numpy as jnp
from jax import lax
from jax.experimental import pallas as pl
from jax.experimental.pallas import tpu as pltpu
from jax.experimental.pallas import tpu_sc as plsc

F = 64
_EB = 400
_NT = 16
_LANES = 16


def _make_agg(V, E, nseg, R, n_chunks):
    epw = E // _NT
    assert epw % _EB == 0
    nb = epw // _EB
    ACC = R + 128
    ZR = ACC // _NT
    OS = R // _NT
    assert R % 128 == 0 and n_chunks % 2 == 0
    npc = n_chunks // 2
    nseg_pad = R * n_chunks
    mesh = plsc.VectorSubcoreMesh(core_axis_name="c", subcore_axis_name="s")

    @functools.partial(
        pl.kernel,
        mesh=mesh,
        compiler_params=pltpu.CompilerParams(use_tc_tiling_on_sc=False),
        out_type=jax.ShapeDtypeStruct((nseg_pad, F), jnp.float32),
        scratch_types=[
            pltpu.VMEM_SHARED((ACC, F), jnp.float32),
            pltpu.VMEM((_EB,), jnp.int32),
            pltpu.VMEM((_EB,), jnp.int32),
            pltpu.VMEM((_EB,), jnp.int32),
            pltpu.VMEM((_EB, F), jnp.float32),
            pltpu.SemaphoreType.DMA,
        ],
    )
    def k(table, src, dst, zrows, out, acc, src_v, dst_v, sidx_v, rows_v, sem):
        c = lax.axis_index("c")
        s = lax.axis_index("s")
        tbase = s * epw
        for kk in range(npc):
            cid = c + 2 * kk
            base = cid * R
            pltpu.sync_copy(zrows, acc.at[pl.ds(s * ZR, ZR)])
            plsc.subcore_barrier()

            def batch(b, _):
                eoff = tbase + b * _EB
                pltpu.sync_copy(src.at[pl.ds(eoff, _EB)], src_v)
                pltpu.sync_copy(dst.at[pl.ds(eoff, _EB)], dst_v)
                pltpu.async_copy(table.at[src_v], rows_v, sem).wait()

                def vloop(i, _):
                    d = dst_v[pl.ds(i * _LANES, _LANES)]
                    loc = d - base
                    inb = (loc >= 0) & (loc < R)
                    sidx_v[pl.ds(i * _LANES, _LANES)] = jnp.where(inb, loc, R)
                    return 0

                lax.fori_loop(0, _EB // _LANES, vloop, 0)
                pltpu.sync_copy(rows_v, acc.at[sidx_v], add=True)
                return 0

            lax.fori_loop(0, nb, batch, 0)
            plsc.subcore_barrier()
            pltpu.sync_copy(acc.at[pl.ds(s * OS, OS)],
                            out.at[pl.ds(base + s * OS, OS)])
            plsc.subcore_barrier()

    zrows_const = jnp.zeros((ZR, F), jnp.float32)

    def run(table, src, dst):
        return k(table, src, dst, zrows_const)

    return run


_BLK = 512


def _row_grid(M):
    return (pl.cdiv(M, _BLK),)


def _mm_bias(x, W, b2):
    M, K = x.shape
    N = W.shape[1]

    def body(x_ref, w_ref, b_ref, o_ref):
        o_ref[...] = (
            jnp.dot(x_ref[...], w_ref[...], preferred_element_type=jnp.float32)
            + b_ref[...]
        )

    return pl.pallas_call(
        body,
        grid=_row_grid(M),
        in_specs=[
            pl.BlockSpec((_BLK, K), lambda i: (i, 0)),
            pl.BlockSpec((K, N), lambda i: (0, 0)),
            pl.BlockSpec((1, N), lambda i: (0, 0)),
        ],
        out_specs=pl.BlockSpec((_BLK, N), lambda i: (i, 0)),
        out_shape=jax.ShapeDtypeStruct((M, N), jnp.float32),
    )(x, W, b2)


def _bond_update(S, cnt, h, Wr, b2):
    M = h.shape[0]

    def body(s_ref, c_ref, h_ref, w_ref, b_ref, o_ref):
        mean = s_ref[...] / jnp.maximum(c_ref[...], 1.0)
        o_ref[...] = jnp.maximum(
            mean
            + jnp.dot(h_ref[...], w_ref[...], preferred_element_type=jnp.float32)
            + b_ref[...],
            0.0,
        )

    return pl.pallas_call(
        body,
        grid=_row_grid(M),
        in_specs=[
            pl.BlockSpec((_BLK, F), lambda i: (i, 0)),
            pl.BlockSpec((_BLK, F), lambda i: (i, 0)),
            pl.BlockSpec((_BLK, F), lambda i: (i, 0)),
            pl.BlockSpec((F, F), lambda i: (0, 0)),
            pl.BlockSpec((1, F), lambda i: (0, 0)),
        ],
        out_specs=pl.BlockSpec((_BLK, F), lambda i: (i, 0)),
        out_shape=jax.ShapeDtypeStruct((M, F), jnp.float32),
    )(S, cnt, h, Wr, b2)


def _atom_update(S, cnt, h, Wn, Wr, b2):
    M = h.shape[0]

    def body(s_ref, c_ref, h_ref, wn_ref, wr_ref, b_ref, o_ref):
        mean = s_ref[...] / jnp.maximum(c_ref[...], 1.0)
        o_ref[...] = jnp.maximum(
            jnp.dot(mean, wn_ref[...], preferred_element_type=jnp.float32)
            + jnp.dot(h_ref[...], wr_ref[...], preferred_element_type=jnp.float32)
            + b_ref[...],
            0.0,
        )

    return pl.pallas_call(
        body,
        grid=_row_grid(M),
        in_specs=[
            pl.BlockSpec((_BLK, F), lambda i: (i, 0)),
            pl.BlockSpec((_BLK, F), lambda i: (i, 0)),
            pl.BlockSpec((_BLK, F), lambda i: (i, 0)),
            pl.BlockSpec((F, F), lambda i: (0, 0)),
            pl.BlockSpec((F, F), lambda i: (0, 0)),
            pl.BlockSpec((1, F), lambda i: (0, 0)),
        ],
        out_specs=pl.BlockSpec((_BLK, F), lambda i: (i, 0)),
        out_shape=jax.ShapeDtypeStruct((M, F), jnp.float32),
    )(S, cnt, h, Wn, Wr, b2)


def _mlp_head(h, W1, b1, W2, b2):
    M = h.shape[0]

    def body(h_ref, w1_ref, b1_ref, w2_ref, b2_ref, o_ref):
        t = jnp.maximum(
            jnp.dot(h_ref[...], w1_ref[...], preferred_element_type=jnp.float32)
            + b1_ref[...],
            0.0,
        )
        o_ref[...] = (
            jnp.dot(t, w2_ref[...], preferred_element_type=jnp.float32) + b2_ref[...]
        )

    return pl.pallas_call(
        body,
        grid=_row_grid(M),
        in_specs=[
            pl.BlockSpec((_BLK, F), lambda i: (i, 0)),
            pl.BlockSpec((F, F), lambda i: (0, 0)),
            pl.BlockSpec((1, F), lambda i: (0, 0)),
            pl.BlockSpec((F, 1), lambda i: (0, 0)),
            pl.BlockSpec((1, 1), lambda i: (0, 0)),
        ],
        out_specs=pl.BlockSpec((_BLK, 1), lambda i: (i, 0)),
        out_shape=jax.ShapeDtypeStruct((M, 1), jnp.float32),
    )(h, W1, b1, W2, b2)


_N_ATOMS = 10000
_N_BONDS = 160000
_E = 320000

_agg_to_bonds = None
_agg_to_atoms = None


def _get_aggs():
    global _agg_to_bonds, _agg_to_atoms
    if _agg_to_bonds is None:
        _agg_to_bonds = _make_agg(_N_ATOMS, _E, _N_BONDS, 20096, 8)
        _agg_to_atoms = _make_agg(_N_BONDS, _E, _N_ATOMS, 5120, 2)
    return _agg_to_bonds, _agg_to_atoms


def kernel(x_atoms, x_bonds, ei_a2b_src, ei_a2b_dst, ei_b2a_src, ei_b2a_dst,
           W_ua, b_ua, W_ub, b_ub,
           Wn_b0, Wr_b0, bb0, Wn_a0, Wr_a0, ba0,
           Wn_b1, Wr_b1, bb1, Wn_a1, Wr_a1, ba1,
           Wn_b2, Wr_b2, bb2, Wn_a2, Wr_a2, ba2,
           Wp1, bp1, Wp2, bp2):
    agg_b, agg_a = _get_aggs()
    r2 = lambda v: v.reshape(1, -1)

    ha = _mm_bias(x_atoms, W_ua, r2(b_ua))
    hb = _mm_bias(x_bonds, W_ub, r2(b_ub))

    ones_a = jnp.ones((_N_ATOMS, F), jnp.float32)
    ones_b = jnp.ones((_N_BONDS, F), jnp.float32)
    cnt_b = agg_b(ones_a, ei_a2b_src, ei_a2b_dst)[:_N_BONDS]
    cnt_a = agg_a(ones_b, ei_b2a_src, ei_b2a_dst)[:_N_ATOMS]

    layers = [(Wn_b0, Wr_b0, bb0, Wn_a0, Wr_a0, ba0),
              (Wn_b1, Wr_b1, bb1, Wn_a1, Wr_a1, ba1),
              (Wn_b2, Wr_b2, bb2, Wn_a2, Wr_a2, ba2)]
    zb = jnp.zeros((1, F), jnp.float32)
    for (Wnb, Wrb, bb, Wna, Wra, ba) in layers:
        pa = _mm_bias(ha, Wnb, zb)
        Sb = agg_b(pa, ei_a2b_src, ei_a2b_dst)[:_N_BONDS]
        Sa = agg_a(hb, ei_b2a_src, ei_b2a_dst)[:_N_ATOMS]
        hb_new = _bond_update(Sb, cnt_b, hb, Wrb, r2(bb))
        ha_new = _atom_update(Sa, cnt_a, ha, Wna, Wra, r2(ba))
        ha, hb = ha_new, hb_new

    out = _mlp_head(hb, Wp1, r2(bp1), Wp2, r2(bp2))
    return jnp.squeeze(out)


# speedup vs baseline: 1.4402x; 1.2533x over previous
import functools

import jax
import jax.numpy as jnp
from jax import lax
from jax.experimental import pallas as pl
from jax.experimental.pallas import tpu as pltpu
from jax.experimental.pallas import tpu_sc as plsc

F = 64
_EB = 400
_NT = 16
_LANES = 16


def _make_agg(V, E, nseg, R, n_chunks, split_edges, Fw, do_gather):
    n_scan_tiles = 32 if split_edges else _NT
    epw = E // n_scan_tiles
    assert epw % _EB == 0
    nb = epw // _EB
    ACC = R + 128
    ZR = ACC // _NT
    OS = R // _NT
    assert R % 128 == 0 and (split_edges or n_chunks % 2 == 0)
    npc = n_chunks if split_edges else n_chunks // 2
    nseg_pad = R * n_chunks
    out_shape = ((2, nseg_pad, Fw) if split_edges else (nseg_pad, Fw))
    mesh = plsc.VectorSubcoreMesh(core_axis_name="c", subcore_axis_name="s")

    @functools.partial(
        pl.kernel,
        mesh=mesh,
        compiler_params=pltpu.CompilerParams(use_tc_tiling_on_sc=False),
        out_type=jax.ShapeDtypeStruct(out_shape, jnp.float32),
        scratch_types=[
            pltpu.VMEM_SHARED((ACC, Fw), jnp.float32),
            pltpu.VMEM((_EB,), jnp.int32),
            pltpu.VMEM((_EB,), jnp.int32),
            pltpu.VMEM((_EB,), jnp.int32),
            pltpu.VMEM((_EB, Fw), jnp.float32),
            pltpu.SemaphoreType.DMA,
        ],
    )
    def k(table, src, dst, zrows, out, acc, src_v, dst_v, sidx_v, rows_v, sem):
        c = lax.axis_index("c")
        s = lax.axis_index("s")
        tbase = (s * 2 + c) * epw if split_edges else s * epw
        if not do_gather:
            pltpu.sync_copy(table, rows_v)
        for kk in range(npc):
            base = kk * R if split_edges else (c + 2 * kk) * R
            pltpu.sync_copy(zrows, acc.at[pl.ds(s * ZR, ZR)])
            plsc.subcore_barrier()

            def batch(b, _):
                eoff = tbase + b * _EB
                if do_gather:
                    pltpu.sync_copy(src.at[pl.ds(eoff, _EB)], src_v)
                pltpu.sync_copy(dst.at[pl.ds(eoff, _EB)], dst_v)
                if do_gather:
                    pltpu.async_copy(table.at[src_v], rows_v, sem).wait()

                def vloop(i, _):
                    d = dst_v[pl.ds(i * _LANES, _LANES)]
                    loc = d - base
                    inb = (loc >= 0) & (loc < R)
                    sidx_v[pl.ds(i * _LANES, _LANES)] = jnp.where(inb, loc, R)
                    return 0

                lax.fori_loop(0, _EB // _LANES, vloop, 0)
                pltpu.sync_copy(rows_v, acc.at[sidx_v], add=True)
                return 0

            lax.fori_loop(0, nb, batch, 0)
            plsc.subcore_barrier()
            if split_edges:
                pltpu.sync_copy(acc.at[pl.ds(s * OS, OS)],
                                out.at[c, pl.ds(base + s * OS, OS)])
            else:
                pltpu.sync_copy(acc.at[pl.ds(s * OS, OS)],
                                out.at[pl.ds(base + s * OS, OS)])
            plsc.subcore_barrier()

    zrows_const = jnp.zeros((ZR, Fw), jnp.float32)

    def run(table, src, dst):
        return k(table, src, dst, zrows_const)

    return run


_BLK = 512


def _row_grid(M):
    return (pl.cdiv(M, _BLK),)


def _mm_bias(x, W, b2):
    M, K = x.shape
    N = W.shape[1]

    def body(x_ref, w_ref, b_ref, o_ref):
        o_ref[...] = (
            jnp.dot(x_ref[...], w_ref[...], preferred_element_type=jnp.float32)
            + b_ref[...]
        )

    return pl.pallas_call(
        body,
        grid=_row_grid(M),
        in_specs=[
            pl.BlockSpec((_BLK, K), lambda i: (i, 0)),
            pl.BlockSpec((K, N), lambda i: (0, 0)),
            pl.BlockSpec((1, N), lambda i: (0, 0)),
        ],
        out_specs=pl.BlockSpec((_BLK, N), lambda i: (i, 0)),
        out_shape=jax.ShapeDtypeStruct((M, N), jnp.float32),
    )(x, W, b2)


def _bond_update(S, cnt, h, Wr, b2):
    M = h.shape[0]

    def body(s_ref, c_ref, h_ref, w_ref, b_ref, o_ref):
        denom = jnp.maximum(c_ref[...][:, 0:1], 1.0)
        o_ref[...] = jnp.maximum(
            s_ref[...] / denom
            + jnp.dot(h_ref[...], w_ref[...], preferred_element_type=jnp.float32)
            + b_ref[...],
            0.0,
        )

    return pl.pallas_call(
        body,
        grid=_row_grid(M),
        in_specs=[
            pl.BlockSpec((_BLK, F), lambda i: (i, 0)),
            pl.BlockSpec((_BLK, 16), lambda i: (i, 0)),
            pl.BlockSpec((_BLK, F), lambda i: (i, 0)),
            pl.BlockSpec((F, F), lambda i: (0, 0)),
            pl.BlockSpec((1, F), lambda i: (0, 0)),
        ],
        out_specs=pl.BlockSpec((_BLK, F), lambda i: (i, 0)),
        out_shape=jax.ShapeDtypeStruct((M, F), jnp.float32),
    )(S, cnt, h, Wr, b2)


def _atom_update(S2, cnt2, h, Wn, Wr, b2):
    M = h.shape[0]

    def body(s_ref, c_ref, h_ref, wn_ref, wr_ref, b_ref, o_ref):
        Ssum = s_ref[0] + s_ref[1]
        denom = jnp.maximum(c_ref[0][:, 0:1] + c_ref[1][:, 0:1], 1.0)
        mean = Ssum / denom
        o_ref[...] = jnp.maximum(
            jnp.dot(mean, wn_ref[...], preferred_element_type=jnp.float32)
            + jnp.dot(h_ref[...], wr_ref[...], preferred_element_type=jnp.float32)
            + b_ref[...],
            0.0,
        )

    return pl.pallas_call(
        body,
        grid=_row_grid(M),
        in_specs=[
            pl.BlockSpec((2, _BLK, F), lambda i: (0, i, 0)),
            pl.BlockSpec((2, _BLK, 16), lambda i: (0, i, 0)),
            pl.BlockSpec((_BLK, F), lambda i: (i, 0)),
            pl.BlockSpec((F, F), lambda i: (0, 0)),
            pl.BlockSpec((F, F), lambda i: (0, 0)),
            pl.BlockSpec((1, F), lambda i: (0, 0)),
        ],
        out_specs=pl.BlockSpec((_BLK, F), lambda i: (i, 0)),
        out_shape=jax.ShapeDtypeStruct((M, F), jnp.float32),
    )(S2, cnt2, h, Wn, Wr, b2)


def _mlp_head(h, W1, b1, W2, b2):
    M = h.shape[0]

    def body(h_ref, w1_ref, b1_ref, w2_ref, b2_ref, o_ref):
        t = jnp.maximum(
            jnp.dot(h_ref[...], w1_ref[...], preferred_element_type=jnp.float32)
            + b1_ref[...],
            0.0,
        )
        o_ref[...] = (
            jnp.dot(t, w2_ref[...], preferred_element_type=jnp.float32) + b2_ref[...]
        )

    return pl.pallas_call(
        body,
        grid=_row_grid(M),
        in_specs=[
            pl.BlockSpec((_BLK, F), lambda i: (i, 0)),
            pl.BlockSpec((F, F), lambda i: (0, 0)),
            pl.BlockSpec((1, F), lambda i: (0, 0)),
            pl.BlockSpec((F, 1), lambda i: (0, 0)),
            pl.BlockSpec((1, 1), lambda i: (0, 0)),
        ],
        out_specs=pl.BlockSpec((_BLK, 1), lambda i: (i, 0)),
        out_shape=jax.ShapeDtypeStruct((M, 1), jnp.float32),
    )(h, W1, b1, W2, b2)


_N_ATOMS = 10000
_N_BONDS = 160000
_E = 320000

_aggs = None


def _get_aggs():
    global _aggs
    if _aggs is None:
        _aggs = (
            _make_agg(_N_ATOMS, _E, _N_BONDS, 20096, 8, False, F, True),
            _make_agg(_N_BONDS, _E, _N_ATOMS, 10240, 1, True, F, True),
            _make_agg(_N_ATOMS, _E, _N_BONDS, 80384, 2, False, 16, False),
            _make_agg(_N_BONDS, _E, _N_ATOMS, 10240, 1, True, 16, False),
        )
    return _aggs


def kernel(x_atoms, x_bonds, ei_a2b_src, ei_a2b_dst, ei_b2a_src, ei_b2a_dst,
           W_ua, b_ua, W_ub, b_ub,
           Wn_b0, Wr_b0, bb0, Wn_a0, Wr_a0, ba0,
           Wn_b1, Wr_b1, bb1, Wn_a1, Wr_a1, ba1,
           Wn_b2, Wr_b2, bb2, Wn_a2, Wr_a2, ba2,
           Wp1, bp1, Wp2, bp2):
    agg_b, agg_a, cnt_kb, cnt_ka = _get_aggs()
    r2 = lambda v: v.reshape(1, -1)

    ha = _mm_bias(x_atoms, W_ua, r2(b_ua))
    hb = _mm_bias(x_bonds, W_ub, r2(b_ub))

    ones16 = jnp.ones((_EB, 16), jnp.float32)
    cnt_b = cnt_kb(ones16, ei_a2b_src, ei_a2b_dst)[:_N_BONDS]
    cnt_a = cnt_ka(ones16, ei_b2a_src, ei_b2a_dst)[:, :_N_ATOMS]

    layers = [(Wn_b0, Wr_b0, bb0, Wn_a0, Wr_a0, ba0),
              (Wn_b1, Wr_b1, bb1, Wn_a1, Wr_a1, ba1),
              (Wn_b2, Wr_b2, bb2, Wn_a2, Wr_a2, ba2)]
    zb = jnp.zeros((1, F), jnp.float32)
    for (Wnb, Wrb, bb, Wna, Wra, ba) in layers:
        pa = _mm_bias(ha, Wnb, zb)
        Sb = agg_b(pa, ei_a2b_src, ei_a2b_dst)[:_N_BONDS]
        Sa = agg_a(hb, ei_b2a_src, ei_b2a_dst)[:, :_N_ATOMS]
        hb_new = _bond_update(Sb, cnt_b, hb, Wrb, r2(bb))
        ha_new = _atom_update(Sa, cnt_a, ha, Wna, Wra, r2(ba))
        ha, hb = ha_new, hb_new

    out = _mlp_head(hb, Wp1, r2(bp1), Wp2, r2(bp2))
    return jnp.squeeze(out)


# speedup vs baseline: 1.7421x; 1.2097x over previous
import functools

import jax
import jax.numpy as jnp
from jax import lax
from jax.experimental import pallas as pl
from jax.experimental.pallas import tpu as pltpu
from jax.experimental.pallas import tpu_sc as plsc

F = 64
_EB = 400
_NT = 16
_LANES = 16


def _make_agg(V, E, nseg, R, n_chunks, split_edges, Fw, do_gather):
    n_scan_tiles = 32 if split_edges else _NT
    epw = E // n_scan_tiles
    assert epw % _EB == 0
    nb = epw // _EB
    ACC = R + 128
    ZR = ACC // _NT
    OS = R // _NT
    assert R % 128 == 0 and (split_edges or n_chunks % 2 == 0)
    npc = n_chunks if split_edges else n_chunks // 2
    nseg_pad = R * n_chunks
    out_shape = ((2, nseg_pad, Fw) if split_edges else (nseg_pad, Fw))
    mesh = plsc.VectorSubcoreMesh(core_axis_name="c", subcore_axis_name="s")

    @functools.partial(
        pl.kernel,
        mesh=mesh,
        compiler_params=pltpu.CompilerParams(use_tc_tiling_on_sc=False),
        out_type=jax.ShapeDtypeStruct(out_shape, jnp.float32),
        scratch_types=[
            pltpu.VMEM_SHARED((ACC, Fw), jnp.float32),
            pltpu.VMEM((_EB,), jnp.int32),
            pltpu.VMEM((_EB,), jnp.int32),
            pltpu.VMEM((_EB,), jnp.int32),
            pltpu.VMEM((_EB, Fw), jnp.float32),
            pltpu.SemaphoreType.DMA,
        ],
    )
    def k(table, src, dst, zrows, out, acc, src_v, dst_v, sidx_v, rows_v, sem):
        c = lax.axis_index("c")
        s = lax.axis_index("s")
        tbase = (s * 2 + c) * epw if split_edges else s * epw
        if not do_gather:
            pltpu.sync_copy(table, rows_v)
        for kk in range(npc):
            base = kk * R if split_edges else (c + 2 * kk) * R
            pltpu.sync_copy(zrows, acc.at[pl.ds(s * ZR, ZR)])
            plsc.subcore_barrier()

            def batch(b, _):
                eoff = tbase + b * _EB
                if do_gather:
                    pltpu.sync_copy(src.at[pl.ds(eoff, _EB)], src_v)
                pltpu.sync_copy(dst.at[pl.ds(eoff, _EB)], dst_v)
                if do_gather:
                    pltpu.async_copy(table.at[src_v], rows_v, sem).wait()

                def vloop(i, _):
                    d = dst_v[pl.ds(i * _LANES, _LANES)]
                    loc = d - base
                    inb = (loc >= 0) & (loc < R)
                    sidx_v[pl.ds(i * _LANES, _LANES)] = jnp.where(inb, loc, R)
                    return 0

                lax.fori_loop(0, _EB // _LANES, vloop, 0)
                pltpu.sync_copy(rows_v, acc.at[sidx_v], add=True)
                return 0

            lax.fori_loop(0, nb, batch, 0)
            plsc.subcore_barrier()
            if split_edges:
                pltpu.sync_copy(acc.at[pl.ds(s * OS, OS)],
                                out.at[c, pl.ds(base + s * OS, OS)])
            else:
                pltpu.sync_copy(acc.at[pl.ds(s * OS, OS)],
                                out.at[pl.ds(base + s * OS, OS)])
            plsc.subcore_barrier()

    zrows_const = jnp.zeros((ZR, Fw), jnp.float32)

    def run(table, src, dst):
        return k(table, src, dst, zrows_const)

    return run


def _make_agg_bonds_fs(E, Rh):
    epw = E // _NT
    nb = epw // _EB
    ACC = Rh + 128
    ZR = ACC // _NT
    OS = Rh // _NT
    assert Rh % 128 == 0 and epw % _EB == 0
    nseg_pad = 2 * Rh
    mesh = plsc.VectorSubcoreMesh(core_axis_name="c", subcore_axis_name="s")

    @functools.partial(
        pl.kernel,
        mesh=mesh,
        compiler_params=pltpu.CompilerParams(use_tc_tiling_on_sc=False),
        out_type=jax.ShapeDtypeStruct((4, nseg_pad, 16), jnp.float32),
        scratch_types=[
            pltpu.VMEM_SHARED((ACC, 16), jnp.float32),
            pltpu.VMEM((_EB,), jnp.int32),
            pltpu.VMEM((_EB,), jnp.int32),
            pltpu.VMEM((_EB,), jnp.int32),
            pltpu.VMEM((_EB, 16), jnp.float32),
            pltpu.SemaphoreType.DMA,
        ],
    )
    def k(t0, t1, t2, t3, src, dst, zrows, out,
          acc, src_v, dst_v, sidx_v, rows_v, sem):
        c = lax.axis_index("c")
        s = lax.axis_index("s")
        tbase = s * epw
        base = c * Rh
        for fc, table in enumerate((t0, t1, t2, t3)):
            pltpu.sync_copy(zrows, acc.at[pl.ds(s * ZR, ZR)])
            plsc.subcore_barrier()

            def batch(b, _):
                eoff = tbase + b * _EB
                pltpu.sync_copy(src.at[pl.ds(eoff, _EB)], src_v)
                pltpu.sync_copy(dst.at[pl.ds(eoff, _EB)], dst_v)
                pltpu.async_copy(table.at[src_v], rows_v, sem).wait()

                def vloop(i, _):
                    d = dst_v[pl.ds(i * _LANES, _LANES)]
                    loc = d - base
                    inb = (loc >= 0) & (loc < Rh)
                    sidx_v[pl.ds(i * _LANES, _LANES)] = jnp.where(inb, loc, Rh)
                    return 0

                lax.fori_loop(0, _EB // _LANES, vloop, 0)
                pltpu.sync_copy(rows_v, acc.at[sidx_v], add=True)
                return 0

            lax.fori_loop(0, nb, batch, 0)
            plsc.subcore_barrier()
            pltpu.sync_copy(acc.at[pl.ds(s * OS, OS)],
                            out.at[fc, pl.ds(base + s * OS, OS)])
            plsc.subcore_barrier()

    zrows_const = jnp.zeros((ZR, 16), jnp.float32)

    def run(table, src, dst):
        ts = [table[:, 16 * j:16 * (j + 1)] for j in range(4)]
        return k(ts[0], ts[1], ts[2], ts[3], src, dst, zrows_const)

    return run


_BLK = 512


def _row_grid(M):
    return (pl.cdiv(M, _BLK),)


def _mm_bias(x, W, b2):
    M, K = x.shape
    N = W.shape[1]

    def body(x_ref, w_ref, b_ref, o_ref):
        o_ref[...] = (
            jnp.dot(x_ref[...], w_ref[...], preferred_element_type=jnp.float32)
            + b_ref[...]
        )

    return pl.pallas_call(
        body,
        grid=_row_grid(M),
        in_specs=[
            pl.BlockSpec((_BLK, K), lambda i: (i, 0)),
            pl.BlockSpec((K, N), lambda i: (0, 0)),
            pl.BlockSpec((1, N), lambda i: (0, 0)),
        ],
        out_specs=pl.BlockSpec((_BLK, N), lambda i: (i, 0)),
        out_shape=jax.ShapeDtypeStruct((M, N), jnp.float32),
    )(x, W, b2)


def _bond_update(S, cnt, h, Wr, b2):
    M = h.shape[0]

    def body(s_ref, c_ref, h_ref, w_ref, b_ref, o_ref):
        denom = jnp.maximum(c_ref[...][:, 0:1], 1.0)
        S = jnp.concatenate([s_ref[j] for j in range(4)], axis=1)
        o_ref[...] = jnp.maximum(
            S / denom
            + jnp.dot(h_ref[...], w_ref[...], preferred_element_type=jnp.float32)
            + b_ref[...],
            0.0,
        )

    return pl.pallas_call(
        body,
        grid=_row_grid(M),
        in_specs=[
            pl.BlockSpec((4, _BLK, 16), lambda i: (0, i, 0)),
            pl.BlockSpec((_BLK, 16), lambda i: (i, 0)),
            pl.BlockSpec((_BLK, F), lambda i: (i, 0)),
            pl.BlockSpec((F, F), lambda i: (0, 0)),
            pl.BlockSpec((1, F), lambda i: (0, 0)),
        ],
        out_specs=pl.BlockSpec((_BLK, F), lambda i: (i, 0)),
        out_shape=jax.ShapeDtypeStruct((M, F), jnp.float32),
    )(S, cnt, h, Wr, b2)


def _atom_update(S2, cnt2, h, Wn, Wr, b2):
    M = h.shape[0]

    def body(s_ref, c_ref, h_ref, wn_ref, wr_ref, b_ref, o_ref):
        Ssum = s_ref[0] + s_ref[1]
        denom = jnp.maximum(c_ref[0][:, 0:1] + c_ref[1][:, 0:1], 1.0)
        mean = Ssum / denom
        o_ref[...] = jnp.maximum(
            jnp.dot(mean, wn_ref[...], preferred_element_type=jnp.float32)
            + jnp.dot(h_ref[...], wr_ref[...], preferred_element_type=jnp.float32)
            + b_ref[...],
            0.0,
        )

    return pl.pallas_call(
        body,
        grid=_row_grid(M),
        in_specs=[
            pl.BlockSpec((2, _BLK, F), lambda i: (0, i, 0)),
            pl.BlockSpec((2, _BLK, 16), lambda i: (0, i, 0)),
            pl.BlockSpec((_BLK, F), lambda i: (i, 0)),
            pl.BlockSpec((F, F), lambda i: (0, 0)),
            pl.BlockSpec((F, F), lambda i: (0, 0)),
            pl.BlockSpec((1, F), lambda i: (0, 0)),
        ],
        out_specs=pl.BlockSpec((_BLK, F), lambda i: (i, 0)),
        out_shape=jax.ShapeDtypeStruct((M, F), jnp.float32),
    )(S2, cnt2, h, Wn, Wr, b2)


def _mlp_head(h, W1, b1, W2, b2):
    M = h.shape[0]

    def body(h_ref, w1_ref, b1_ref, w2_ref, b2_ref, o_ref):
        t = jnp.maximum(
            jnp.dot(h_ref[...], w1_ref[...], preferred_element_type=jnp.float32)
            + b1_ref[...],
            0.0,
        )
        o_ref[...] = (
            jnp.dot(t, w2_ref[...], preferred_element_type=jnp.float32) + b2_ref[...]
        )

    return pl.pallas_call(
        body,
        grid=_row_grid(M),
        in_specs=[
            pl.BlockSpec((_BLK, F), lambda i: (i, 0)),
            pl.BlockSpec((F, F), lambda i: (0, 0)),
            pl.BlockSpec((1, F), lambda i: (0, 0)),
            pl.BlockSpec((F, 1), lambda i: (0, 0)),
            pl.BlockSpec((1, 1), lambda i: (0, 0)),
        ],
        out_specs=pl.BlockSpec((_BLK, 1), lambda i: (i, 0)),
        out_shape=jax.ShapeDtypeStruct((M, 1), jnp.float32),
    )(h, W1, b1, W2, b2)


_N_ATOMS = 10000
_N_BONDS = 160000
_E = 320000

_aggs = None


def _get_aggs():
    global _aggs
    if _aggs is None:
        _aggs = (
            _make_agg_bonds_fs(_E, 80384),
            _make_agg(_N_BONDS, _E, _N_ATOMS, 10240, 1, True, F, True),
            _make_agg(_N_ATOMS, _E, _N_BONDS, 80384, 2, False, 16, False),
            _make_agg(_N_BONDS, _E, _N_ATOMS, 10240, 1, True, 16, False),
        )
    return _aggs


def kernel(x_atoms, x_bonds, ei_a2b_src, ei_a2b_dst, ei_b2a_src, ei_b2a_dst,
           W_ua, b_ua, W_ub, b_ub,
           Wn_b0, Wr_b0, bb0, Wn_a0, Wr_a0, ba0,
           Wn_b1, Wr_b1, bb1, Wn_a1, Wr_a1, ba1,
           Wn_b2, Wr_b2, bb2, Wn_a2, Wr_a2, ba2,
           Wp1, bp1, Wp2, bp2):
    agg_b, agg_a, cnt_kb, cnt_ka = _get_aggs()
    r2 = lambda v: v.reshape(1, -1)

    ha = _mm_bias(x_atoms, W_ua, r2(b_ua))
    hb = _mm_bias(x_bonds, W_ub, r2(b_ub))

    ones16 = jnp.ones((_EB, 16), jnp.float32)
    cnt_b = cnt_kb(ones16, ei_a2b_src, ei_a2b_dst)[:_N_BONDS]
    cnt_a = cnt_ka(ones16, ei_b2a_src, ei_b2a_dst)[:, :_N_ATOMS]

    layers = [(Wn_b0, Wr_b0, bb0, Wn_a0, Wr_a0, ba0),
              (Wn_b1, Wr_b1, bb1, Wn_a1, Wr_a1, ba1),
              (Wn_b2, Wr_b2, bb2, Wn_a2, Wr_a2, ba2)]
    zb = jnp.zeros((1, F), jnp.float32)
    for (Wnb, Wrb, bb, Wna, Wra, ba) in layers:
        pa = _mm_bias(ha, Wnb, zb)
        Sb = agg_b(pa, ei_a2b_src, ei_a2b_dst)[:, :_N_BONDS]
        Sa = agg_a(hb, ei_b2a_src, ei_b2a_dst)[:, :_N_ATOMS]
        hb_new = _bond_update(Sb, cnt_b, hb, Wrb, r2(bb))
        ha_new = _atom_update(Sa, cnt_a, ha, Wna, Wra, r2(ba))
        ha, hb = ha_new, hb_new

    out = _mlp_head(hb, Wp1, r2(bp1), Wp2, r2(bp2))
    return jnp.squeeze(out)


# speedup vs baseline: 1.7632x; 1.0121x over previous
import functools

import jax
import jax.numpy as jnp
from jax import lax
from jax.experimental import pallas as pl
from jax.experimental.pallas import tpu as pltpu
from jax.experimental.pallas import tpu_sc as plsc

F = 64
_EB = 400
_NT = 16
_LANES = 16


def _make_agg(V, E, nseg, R, n_chunks, split_edges, Fw, do_gather, eb=_EB):
    n_scan_tiles = 32 if split_edges else _NT
    epw = E // n_scan_tiles
    assert epw % eb == 0 and eb % 8 == 0
    nvl = (eb + _LANES - 1) // _LANES
    nb = epw // eb
    ACC = R + 128
    ZR = ACC // _NT
    OS = R // _NT
    assert R % 128 == 0 and (split_edges or n_chunks % 2 == 0)
    npc = n_chunks if split_edges else n_chunks // 2
    nseg_pad = R * n_chunks
    out_shape = ((2, nseg_pad, Fw) if split_edges else (nseg_pad, Fw))
    mesh = plsc.VectorSubcoreMesh(core_axis_name="c", subcore_axis_name="s")

    @functools.partial(
        pl.kernel,
        mesh=mesh,
        compiler_params=pltpu.CompilerParams(use_tc_tiling_on_sc=False),
        out_type=jax.ShapeDtypeStruct(out_shape, jnp.float32),
        scratch_types=[
            pltpu.VMEM_SHARED((ACC, Fw), jnp.float32),
            pltpu.VMEM((eb,), jnp.int32),
            pltpu.VMEM((eb,), jnp.int32),
            pltpu.VMEM((eb,), jnp.int32),
            pltpu.VMEM((eb, Fw), jnp.float32),
            pltpu.SemaphoreType.DMA,
        ],
    )
    def k(table, src, dst, zrows, out, acc, src_v, dst_v, sidx_v, rows_v, sem):
        c = lax.axis_index("c")
        s = lax.axis_index("s")
        tbase = (s * 2 + c) * epw if split_edges else s * epw
        if not do_gather:
            pltpu.sync_copy(table, rows_v)
        for kk in range(npc):
            base = kk * R if split_edges else (c + 2 * kk) * R
            pltpu.sync_copy(zrows, acc.at[pl.ds(s * ZR, ZR)])
            plsc.subcore_barrier()

            def batch(b, _):
                eoff = tbase + b * eb
                if do_gather:
                    pltpu.sync_copy(src.at[pl.ds(eoff, eb)], src_v)
                pltpu.sync_copy(dst.at[pl.ds(eoff, eb)], dst_v)
                cp = None
                if do_gather:
                    cp = pltpu.async_copy(table.at[src_v], rows_v, sem)

                def vloop(i, _):
                    o = jnp.minimum(i * _LANES, eb - _LANES)
                    d = dst_v[pl.ds(o, _LANES)]
                    loc = d - base
                    inb = (loc >= 0) & (loc < R)
                    sidx_v[pl.ds(o, _LANES)] = jnp.where(inb, loc, R)
                    return 0

                lax.fori_loop(0, nvl, vloop, 0)
                if cp is not None:
                    cp.wait()
                pltpu.sync_copy(rows_v, acc.at[sidx_v], add=True)
                return 0

            lax.fori_loop(0, nb, batch, 0)
            plsc.subcore_barrier()
            if split_edges:
                pltpu.sync_copy(acc.at[pl.ds(s * OS, OS)],
                                out.at[c, pl.ds(base + s * OS, OS)])
            else:
                pltpu.sync_copy(acc.at[pl.ds(s * OS, OS)],
                                out.at[pl.ds(base + s * OS, OS)])
            plsc.subcore_barrier()

    zrows_const = jnp.zeros((ZR, Fw), jnp.float32)

    def run(table, src, dst):
        return k(table, src, dst, zrows_const)

    return run


def _make_agg_bonds_fs(E, Rh, eb=_EB):
    epw = E // _NT
    nb = epw // eb
    ACC = Rh + 128
    ZR = ACC // _NT
    OS = Rh // _NT
    assert Rh % 128 == 0 and epw % eb == 0 and eb % _LANES == 0
    nseg_pad = 2 * Rh
    mesh = plsc.VectorSubcoreMesh(core_axis_name="c", subcore_axis_name="s")

    @functools.partial(
        pl.kernel,
        mesh=mesh,
        compiler_params=pltpu.CompilerParams(use_tc_tiling_on_sc=False),
        out_type=jax.ShapeDtypeStruct((4, nseg_pad, 16), jnp.float32),
        scratch_types=[
            pltpu.VMEM_SHARED((ACC, 16), jnp.float32),
            pltpu.VMEM((eb,), jnp.int32),
            pltpu.VMEM((eb,), jnp.int32),
            pltpu.VMEM((eb,), jnp.int32),
            pltpu.VMEM((eb, 16), jnp.float32),
            pltpu.SemaphoreType.DMA,
        ],
    )
    def k(t0, t1, t2, t3, src, dst, zrows, out,
          acc, src_v, dst_v, sidx_v, rows_v, sem):
        c = lax.axis_index("c")
        s = lax.axis_index("s")
        tbase = s * epw
        base = c * Rh
        for fc, table in enumerate((t0, t1, t2, t3)):
            pltpu.sync_copy(zrows, acc.at[pl.ds(s * ZR, ZR)])
            plsc.subcore_barrier()

            def batch(b, _):
                eoff = tbase + b * eb
                pltpu.sync_copy(src.at[pl.ds(eoff, eb)], src_v)
                pltpu.sync_copy(dst.at[pl.ds(eoff, eb)], dst_v)
                cp = pltpu.async_copy(table.at[src_v], rows_v, sem)

                def vloop(i, _):
                    d = dst_v[pl.ds(i * _LANES, _LANES)]
                    loc = d - base
                    inb = (loc >= 0) & (loc < Rh)
                    sidx_v[pl.ds(i * _LANES, _LANES)] = jnp.where(inb, loc, Rh)
                    return 0

                lax.fori_loop(0, eb // _LANES, vloop, 0)
                cp.wait()
                pltpu.sync_copy(rows_v, acc.at[sidx_v], add=True)
                return 0

            lax.fori_loop(0, nb, batch, 0)
            plsc.subcore_barrier()
            pltpu.sync_copy(acc.at[pl.ds(s * OS, OS)],
                            out.at[fc, pl.ds(base + s * OS, OS)])
            plsc.subcore_barrier()

    zrows_const = jnp.zeros((ZR, 16), jnp.float32)

    def run(table, src, dst):
        ts = [table[:, 16 * j:16 * (j + 1)] for j in range(4)]
        return k(ts[0], ts[1], ts[2], ts[3], src, dst, zrows_const)

    return run


_BLK = 512


def _row_grid(M):
    return (pl.cdiv(M, _BLK),)


def _mm_bias(x, W, b2):
    M, K = x.shape
    N = W.shape[1]

    def body(x_ref, w_ref, b_ref, o_ref):
        o_ref[...] = (
            jnp.dot(x_ref[...], w_ref[...], preferred_element_type=jnp.float32)
            + b_ref[...]
        )

    return pl.pallas_call(
        body,
        grid=_row_grid(M),
        in_specs=[
            pl.BlockSpec((_BLK, K), lambda i: (i, 0)),
            pl.BlockSpec((K, N), lambda i: (0, 0)),
            pl.BlockSpec((1, N), lambda i: (0, 0)),
        ],
        out_specs=pl.BlockSpec((_BLK, N), lambda i: (i, 0)),
        out_shape=jax.ShapeDtypeStruct((M, N), jnp.float32),
    )(x, W, b2)


def _bond_update(S, cnt, h, Wr, b2):
    M = h.shape[0]

    def body(s_ref, c_ref, h_ref, w_ref, b_ref, o_ref):
        denom = jnp.maximum(c_ref[...][:, 0:1], 1.0)
        S = jnp.concatenate([s_ref[j] for j in range(4)], axis=1)
        o_ref[...] = jnp.maximum(
            S / denom
            + jnp.dot(h_ref[...], w_ref[...], preferred_element_type=jnp.float32)
            + b_ref[...],
            0.0,
        )

    return pl.pallas_call(
        body,
        grid=_row_grid(M),
        in_specs=[
            pl.BlockSpec((4, _BLK, 16), lambda i: (0, i, 0)),
            pl.BlockSpec((_BLK, 16), lambda i: (i, 0)),
            pl.BlockSpec((_BLK, F), lambda i: (i, 0)),
            pl.BlockSpec((F, F), lambda i: (0, 0)),
            pl.BlockSpec((1, F), lambda i: (0, 0)),
        ],
        out_specs=pl.BlockSpec((_BLK, F), lambda i: (i, 0)),
        out_shape=jax.ShapeDtypeStruct((M, F), jnp.float32),
    )(S, cnt, h, Wr, b2)


def _atom_update(S2, cnt2, h, Wn, Wr, b2):
    M = h.shape[0]

    def body(s_ref, c_ref, h_ref, wn_ref, wr_ref, b_ref, o_ref):
        Ssum = s_ref[0] + s_ref[1]
        denom = jnp.maximum(c_ref[0][:, 0:1] + c_ref[1][:, 0:1], 1.0)
        mean = Ssum / denom
        o_ref[...] = jnp.maximum(
            jnp.dot(mean, wn_ref[...], preferred_element_type=jnp.float32)
            + jnp.dot(h_ref[...], wr_ref[...], preferred_element_type=jnp.float32)
            + b_ref[...],
            0.0,
        )

    return pl.pallas_call(
        body,
        grid=_row_grid(M),
        in_specs=[
            pl.BlockSpec((2, _BLK, F), lambda i: (0, i, 0)),
            pl.BlockSpec((2, _BLK, 16), lambda i: (0, i, 0)),
            pl.BlockSpec((_BLK, F), lambda i: (i, 0)),
            pl.BlockSpec((F, F), lambda i: (0, 0)),
            pl.BlockSpec((F, F), lambda i: (0, 0)),
            pl.BlockSpec((1, F), lambda i: (0, 0)),
        ],
        out_specs=pl.BlockSpec((_BLK, F), lambda i: (i, 0)),
        out_shape=jax.ShapeDtypeStruct((M, F), jnp.float32),
    )(S2, cnt2, h, Wn, Wr, b2)


def _mlp_head(h, W1, b1, W2, b2):
    M = h.shape[0]

    def body(h_ref, w1_ref, b1_ref, w2_ref, b2_ref, o_ref):
        t = jnp.maximum(
            jnp.dot(h_ref[...], w1_ref[...], preferred_element_type=jnp.float32)
            + b1_ref[...],
            0.0,
        )
        o_ref[...] = (
            jnp.dot(t, w2_ref[...], preferred_element_type=jnp.float32) + b2_ref[...]
        )

    return pl.pallas_call(
        body,
        grid=_row_grid(M),
        in_specs=[
            pl.BlockSpec((_BLK, F), lambda i: (i, 0)),
            pl.BlockSpec((F, F), lambda i: (0, 0)),
            pl.BlockSpec((1, F), lambda i: (0, 0)),
            pl.BlockSpec((F, 1), lambda i: (0, 0)),
            pl.BlockSpec((1, 1), lambda i: (0, 0)),
        ],
        out_specs=pl.BlockSpec((_BLK, 1), lambda i: (i, 0)),
        out_shape=jax.ShapeDtypeStruct((M, 1), jnp.float32),
    )(h, W1, b1, W2, b2)


_N_ATOMS = 10000
_N_BONDS = 160000
_E = 320000

_aggs = None


def _get_aggs():
    global _aggs
    if _aggs is None:
        _aggs = (
            _make_agg_bonds_fs(_E, 80384, eb=2000),
            _make_agg(_N_BONDS, _E, _N_ATOMS, 10240, 1, True, F, True, eb=1000),
            _make_agg(_N_ATOMS, _E, _N_BONDS, 80384, 2, False, 16, False, eb=2000),
            _make_agg(_N_BONDS, _E, _N_ATOMS, 10240, 1, True, 16, False, eb=2000),
        )
    return _aggs


def kernel(x_atoms, x_bonds, ei_a2b_src, ei_a2b_dst, ei_b2a_src, ei_b2a_dst,
           W_ua, b_ua, W_ub, b_ub,
           Wn_b0, Wr_b0, bb0, Wn_a0, Wr_a0, ba0,
           Wn_b1, Wr_b1, bb1, Wn_a1, Wr_a1, ba1,
           Wn_b2, Wr_b2, bb2, Wn_a2, Wr_a2, ba2,
           Wp1, bp1, Wp2, bp2):
    agg_b, agg_a, cnt_kb, cnt_ka = _get_aggs()
    r2 = lambda v: v.reshape(1, -1)

    ha = _mm_bias(x_atoms, W_ua, r2(b_ua))
    hb = _mm_bias(x_bonds, W_ub, r2(b_ub))

    ones16 = jnp.ones((2000, 16), jnp.float32)
    cnt_b = cnt_kb(ones16, ei_a2b_src, ei_a2b_dst)[:_N_BONDS]
    cnt_a = cnt_ka(ones16, ei_b2a_src, ei_b2a_dst)[:, :_N_ATOMS]

    layers = [(Wn_b0, Wr_b0, bb0, Wn_a0, Wr_a0, ba0),
              (Wn_b1, Wr_b1, bb1, Wn_a1, Wr_a1, ba1),
              (Wn_b2, Wr_b2, bb2, Wn_a2, Wr_a2, ba2)]
    zb = jnp.zeros((1, F), jnp.float32)
    for (Wnb, Wrb, bb, Wna, Wra, ba) in layers:
        pa = _mm_bias(ha, Wnb, zb)
        Sb = agg_b(pa, ei_a2b_src, ei_a2b_dst)[:, :_N_BONDS]
        Sa = agg_a(hb, ei_b2a_src, ei_b2a_dst)[:, :_N_ATOMS]
        hb_new = _bond_update(Sb, cnt_b, hb, Wrb, r2(bb))
        ha_new = _atom_update(Sa, cnt_a, ha, Wna, Wra, r2(ba))
        ha, hb = ha_new, hb_new

    out = _mlp_head(hb, Wp1, r2(bp1), Wp2, r2(bp2))
    return jnp.squeeze(out)
